# baseline (speedup 1.0000x reference)
import functools

import jax
import jax.numpy as jnp
from jax import lax
from jax.experimental import pallas as pl
from jax.experimental.pallas import tpu as pltpu
from jax.experimental.pallas import tpu_sc as plsc

NU = 50000
NI = 50000
N = NU + NI
D = 64
C = 4
DC = D // C
E2 = 2500000
B = 4096

NCORE = 2
NSUB = 16
NW = NCORE * NSUB

BURST = 1024
E2P = 2621440
PAD = E2P - E2
EPT = E2P // NSUB
NBURST = EPT // BURST
GATHER_SPLIT = 8
SCATTER_SPLIT = 8
DUMMY = N
NP = 100096
WROWS = NP // NSUB

DEG_SLICE = 6400
DEG_ROWS = NSUB * DEG_SLICE
DEG_BURST = 1024
DEG_EPT = E2P // NW
DEG_NBURST = DEG_EPT // DEG_BURST

FLATR = C * NP * DC // 128


def _mesh():
    return plsc.VectorSubcoreMesh(core_axis_name="c", subcore_axis_name="s",
                                  num_cores=NCORE, num_subcores=NSUB)


_SC_PARAMS = pltpu.CompilerParams(use_tc_tiling_on_sc=False,
                                  needs_layout_passes=False)


@functools.partial(
    pl.kernel,
    out_type=jax.ShapeDtypeStruct((NCORE, NSUB, DEG_SLICE), jnp.float32),
    compiler_params=_SC_PARAMS,
    mesh=_mesh(),
    scratch_types=[
        pltpu.VMEM((DEG_BURST // 128, 128), jnp.int32),
        pltpu.VMEM((128,), jnp.float32),
        pltpu.VMEM((DEG_SLICE,), jnp.float32),
        pltpu.VMEM_SHARED((DEG_ROWS,), jnp.float32),
        pltpu.SemaphoreType.DMA,
    ],
)
def _sc_degree(dst_hbm, deg_hbm, didx, ones, zeros, hist, sem):
    cid = lax.axis_index("c")
    sid = lax.axis_index("s")

    one = jnp.full((16,), 1.0, jnp.float32)
    zero = jnp.zeros((16,), jnp.float32)
    for i in range(128 // 16):
        ones[pl.ds(16 * i, 16)] = one

    @pl.loop(0, DEG_SLICE // 16)
    def _z(i):
        zeros[pl.ds(16 * i, 16)] = zero

    pltpu.sync_copy(zeros, hist.at[pl.ds(sid * DEG_SLICE, DEG_SLICE)])
    plsc.subcore_barrier()

    row0 = (cid * NSUB + sid) * (DEG_EPT // 128)

    @pl.loop(0, DEG_NBURST)
    def _burst(b):
        r = row0 + b * (DEG_BURST // 128)
        pltpu.sync_copy(dst_hbm.at[pl.ds(r, DEG_BURST // 128), :], didx)
        descs = []
        for j in range(DEG_BURST // 128):
            descs.append(
                pltpu.async_copy(ones, hist.at[didx.at[j]], sem, add=True))
        for d in descs:
            d.wait()

    plsc.subcore_barrier()
    pltpu.sync_copy(hist.at[pl.ds(sid * DEG_SLICE, DEG_SLICE)],
                    deg_hbm.at[cid, sid])


@functools.partial(
    pl.kernel,
    out_type=jax.ShapeDtypeStruct((C * NP, DC), jnp.float32),
    compiler_params=_SC_PARAMS,
    mesh=_mesh(),
    scratch_types=[
        pltpu.VMEM((BURST // 128, 128), jnp.int32),
        pltpu.VMEM((BURST // 128, 128), jnp.int32),
        pltpu.VMEM((BURST, DC), jnp.float32),
        pltpu.VMEM((WROWS // 8, DC), jnp.float32),
        pltpu.VMEM_SHARED((NP, DC), jnp.float32),
        pltpu.SemaphoreType.DMA,
        pltpu.SemaphoreType.DMA,
    ],
)
def _sc_layer(t_hbm, s_hbm, d_hbm, r_hbm, sidx, didx, rows, zeros, acc,
              gsem, ssem):
    cid = lax.axis_index("c")
    sid = lax.axis_index("s")

    zero = jnp.zeros((16,), jnp.float32)

    @pl.loop(0, WROWS // 8)
    def _z(i):
        zeros[i, :] = zero

    row0 = sid * (EPT // 128)

    for cc in range(2):
        c = 2 * cid + cc
        off = jnp.full((16,), c * NP, jnp.int32)

        z0 = sid * WROWS
        for zb in range(8):
            pltpu.sync_copy(
                zeros,
                acc.at[pl.ds(z0 + zb * (WROWS // 8), WROWS // 8), :])
        plsc.subcore_barrier()

        @pl.loop(0, NBURST)
        def _burst(b):
            r = row0 + b * (BURST // 128)
            pltpu.sync_copy(s_hbm.at[pl.ds(r, BURST // 128), :], sidx)
            for i in range(BURST // 128):
                for j in range(128 // 16):
                    sl = pl.ds(16 * j, 16)
                    sidx[i, sl] = sidx[i, sl] + off
            gds = []
            for j in range(BURST // 128):
                gds.append(pltpu.async_copy(
                    t_hbm.at[sidx.at[j]],
                    rows.at[pl.ds(128 * j, 128), :], gsem))
            pltpu.sync_copy(d_hbm.at[pl.ds(r, BURST // 128), :], didx)
            for d in gds:
                d.wait()
            sds = []
            for j in range(BURST // 128):
                sds.append(pltpu.async_copy(
                    rows.at[pl.ds(128 * j, 128), :],
                    acc.at[didx.at[j]], ssem, add=True))
            for d in sds:
                d.wait()

        plsc.subcore_barrier()
        w0 = sid * WROWS
        pltpu.sync_copy(acc.at[pl.ds(w0, WROWS), :],
                        r_hbm.at[pl.ds(c * NP + w0, WROWS), :])
        plsc.subcore_barrier()


BPT = B // NW


@functools.partial(
    pl.kernel,
    out_type=jax.ShapeDtypeStruct((B,), jnp.float32),
    compiler_params=_SC_PARAMS,
    mesh=_mesh(),
    scratch_types=[
        pltpu.VMEM((BPT,), jnp.int32),
        pltpu.VMEM((BPT,), jnp.int32),
        pltpu.VMEM((BPT,), jnp.int32),
        pltpu.VMEM((BPT, DC), jnp.float32),
        pltpu.VMEM((BPT, DC), jnp.float32),
        pltpu.VMEM((BPT, DC), jnp.float32),
        pltpu.VMEM((256,), jnp.float32),
        pltpu.VMEM((BPT,), jnp.float32),
        pltpu.SemaphoreType.DMA,
    ],
)
def _sc_score(acc_hbm, uidx_hbm, iidx_hbm, out_hbm, uv, iv, adj, ur, ir, sv,
              tmpf, res, sem):
    cid = lax.axis_index("c")
    sid = lax.axis_index("s")
    wid = sid * NCORE + cid
    base = wid * BPT

    pltpu.sync_copy(uidx_hbm.at[pl.ds(base, BPT)], uv)
    pltpu.sync_copy(iidx_hbm.at[pl.ds(base, BPT)], iv)

    zero = jnp.zeros((16,), jnp.float32)

    @pl.loop(0, BPT)
    def _z(b):
        sv[b, :] = zero

    for c in range(C):
        uoff = jnp.full((16,), c * NP, jnp.int32)
        ioff = jnp.full((16,), c * NP + NU, jnp.int32)
        for j in range(BPT // 16):
            adj[pl.ds(16 * j, 16)] = uv[pl.ds(16 * j, 16)] + uoff
        pltpu.async_copy(acc_hbm.at[adj], ur, sem).wait()
        for j in range(BPT // 16):
            adj[pl.ds(16 * j, 16)] = iv[pl.ds(16 * j, 16)] + ioff
        pltpu.async_copy(acc_hbm.at[adj], ir, sem).wait()

        @pl.loop(0, BPT)
        def _mac(b):
            sv[b, :] = sv[b, :] + ur[b, :] * ir[b, :]

    col = lax.iota(jnp.int32, 16) * 16
    for g in range(BPT // 16):
        for r in range(16):
            plsc.store_scatter(tmpf, [col + r], sv[16 * g + r, :])
        tot = jnp.zeros((16,), jnp.float32)
        for j in range(16):
            tot = tot + tmpf[pl.ds(16 * j, 16)]
        res[pl.ds(16 * g, 16)] = tot * (1.0 / 16.0)

    pltpu.sync_copy(res, out_hbm.at[pl.ds(base, BPT)])


TCBLK = 128


def _tc_init_body(emb_ref, deg_ref, t_ref):
    dinv = lax.rsqrt(jnp.maximum(deg_ref[...], 1.0))
    t_ref[...] = emb_ref[...] * dinv


def _tc_layer_body(r_ref, deg_ref, acc_ref, t_ref, accn_ref):
    dinv = lax.rsqrt(jnp.maximum(deg_ref[...], 1.0))
    h = r_ref[...] * dinv
    accn_ref[...] = acc_ref[...] + h
    t_ref[...] = h * dinv


def _tc_init(embv, degrep):
    grid = FLATR // TCBLK
    bs = pl.BlockSpec((TCBLK, 128), lambda i: (i, 0))
    return pl.pallas_call(
        _tc_init_body,
        grid=(grid,),
        in_specs=[bs, bs],
        out_specs=bs,
        out_shape=jax.ShapeDtypeStruct((FLATR, 128), jnp.float32),
    )(embv, degrep)


def _tc_layer(rv, degrep, accv):
    grid = FLATR // TCBLK
    bs = pl.BlockSpec((TCBLK, 128), lambda i: (i, 0))
    return pl.pallas_call(
        _tc_layer_body,
        grid=(grid,),
        in_specs=[bs, bs, bs],
        out_specs=[bs, bs],
        out_shape=[
            jax.ShapeDtypeStruct((FLATR, 128), jnp.float32),
            jax.ShapeDtypeStruct((FLATR, 128), jnp.float32),
        ],
    )(rv, degrep, accv)


def kernel(user_emb, item_emb, user_idx, item_idx, edge_index):
    u = edge_index[0]
    it = edge_index[1] + NU
    src = jnp.concatenate([u, it])
    dst = jnp.concatenate([it, u])
    src_p = jnp.concatenate([src, jnp.zeros((PAD,), jnp.int32)])
    dst_p = jnp.concatenate([dst, jnp.full((PAD,), DUMMY, jnp.int32)])
    src2d = src_p.reshape(E2P // 128, 128)
    dst2d = dst_p.reshape(E2P // 128, 128)

    embc = (jnp.concatenate([user_emb, item_emb], axis=0)
            .reshape(N, C, DC).transpose(1, 0, 2))
    embc = jnp.pad(embc, ((0, 0), (0, NP - N), (0, 0))).reshape(C * NP, DC)
    embv = embc.reshape(FLATR, 128)

    deg2 = _sc_degree(dst2d)
    deg = (deg2[0] + deg2[1]).reshape(-1)[:NP]
    degrep = jnp.broadcast_to(deg[None, :, None], (C, NP, DC)).reshape(FLATR, 128)

    t = _tc_init(embv, degrep)
    acc = embv
    for _ in range(3):
        r = _sc_layer(t.reshape(C * NP, DC), src2d, dst2d)
        t, acc = _tc_layer(r.reshape(FLATR, 128), degrep, acc)

    return _sc_score(acc.reshape(C * NP, DC), user_idx, item_idx)


# speedup vs baseline: 1.8713x; 1.8713x over previous
import functools

import jax
import jax.numpy as jnp
from jax import lax
from jax.experimental import pallas as pl
from jax.experimental.pallas import tpu as pltpu
from jax.experimental.pallas import tpu_sc as plsc

NU = 50000
NI = 50000
N = NU + NI
D = 64
C = 4
DC = D // C
E2 = 2500000
B = 4096

NCORE = 2
NSUB = 16
NW = NCORE * NSUB

BURST = 1024
E2P = 2506752
PAD = E2P - E2
EPT = E2P // NSUB
NBURST = EPT // BURST
DUMMY = N
NP = 100096
WROWS = NP // NSUB

DEG_SLICE = 6400
DEG_ROWS = NSUB * DEG_SLICE
DEG_BURST = 512
DEG_EPT = E2P // NW
DEG_NBURST = DEG_EPT // DEG_BURST

FLATR = C * NP * DC // 128


def _mesh():
    return plsc.VectorSubcoreMesh(core_axis_name="c", subcore_axis_name="s",
                                  num_cores=NCORE, num_subcores=NSUB)


_SC_PARAMS = pltpu.CompilerParams(use_tc_tiling_on_sc=False,
                                  needs_layout_passes=False)


@functools.partial(
    pl.kernel,
    out_type=jax.ShapeDtypeStruct((NCORE, NSUB, DEG_SLICE), jnp.float32),
    compiler_params=_SC_PARAMS,
    mesh=_mesh(),
    scratch_types=[
        pltpu.VMEM((DEG_BURST // 128, 128), jnp.int32),
        pltpu.VMEM((128,), jnp.float32),
        pltpu.VMEM((DEG_SLICE,), jnp.float32),
        pltpu.VMEM_SHARED((DEG_ROWS,), jnp.float32),
        pltpu.SemaphoreType.DMA,
    ],
)
def _sc_degree(dst_hbm, deg_hbm, didx, ones, zeros, hist, sem):
    cid = lax.axis_index("c")
    sid = lax.axis_index("s")

    one = jnp.full((16,), 1.0, jnp.float32)
    zero = jnp.zeros((16,), jnp.float32)
    for i in range(128 // 16):
        ones[pl.ds(16 * i, 16)] = one

    @pl.loop(0, DEG_SLICE // 16)
    def _z(i):
        zeros[pl.ds(16 * i, 16)] = zero

    pltpu.sync_copy(zeros, hist.at[pl.ds(sid * DEG_SLICE, DEG_SLICE)])
    plsc.subcore_barrier()

    row0 = (cid * NSUB + sid) * (DEG_EPT // 128)

    @pl.loop(0, DEG_NBURST)
    def _burst(b):
        r = row0 + b * (DEG_BURST // 128)
        pltpu.sync_copy(dst_hbm.at[pl.ds(r, DEG_BURST // 128), :], didx)
        descs = []
        for j in range(DEG_BURST // 128):
            descs.append(
                pltpu.async_copy(ones, hist.at[didx.at[j]], sem, add=True))
        for d in descs:
            d.wait()

    plsc.subcore_barrier()
    pltpu.sync_copy(hist.at[pl.ds(sid * DEG_SLICE, DEG_SLICE)],
                    deg_hbm.at[cid, sid])


@functools.partial(
    pl.kernel,
    out_type=jax.ShapeDtypeStruct((C * NP, DC), jnp.float32),
    compiler_params=_SC_PARAMS,
    mesh=_mesh(),
    scratch_types=[
        pltpu.VMEM((BURST // 128, 128), jnp.int32),
        pltpu.VMEM((BURST // 128, 128), jnp.int32),
        pltpu.VMEM((BURST, DC), jnp.float32),
        pltpu.VMEM((WROWS // 8, DC), jnp.float32),
        pltpu.VMEM_SHARED((NP, DC), jnp.float32),
        pltpu.SemaphoreType.DMA,
        pltpu.SemaphoreType.DMA,
    ],
)
def _sc_layer(t_hbm, s_hbm, d_hbm, r_hbm, sidx, didx, rows, zeros, acc,
              gsem, ssem):
    cid = lax.axis_index("c")
    sid = lax.axis_index("s")

    zero = jnp.zeros((16,), jnp.float32)

    @pl.loop(0, WROWS // 8)
    def _z(i):
        zeros[i, :] = zero

    row0 = sid * (EPT // 128)

    for cc in range(2):
        c = 2 * cid + cc
        off = jnp.full((16,), c * NP, jnp.int32)

        z0 = sid * WROWS
        for zb in range(8):
            pltpu.sync_copy(
                zeros,
                acc.at[pl.ds(z0 + zb * (WROWS // 8), WROWS // 8), :])
        plsc.subcore_barrier()

        @pl.loop(0, NBURST)
        def _burst(b):
            r = row0 + b * (BURST // 128)
            pltpu.sync_copy(s_hbm.at[pl.ds(r, BURST // 128), :], sidx)
            for i in range(BURST // 128):
                for j in range(128 // 16):
                    sl = pl.ds(16 * j, 16)
                    sidx[i, sl] = sidx[i, sl] + off
            gds = []
            for j in range(BURST // 128):
                gds.append(pltpu.async_copy(
                    t_hbm.at[sidx.at[j]],
                    rows.at[pl.ds(128 * j, 128), :], gsem))
            pltpu.sync_copy(d_hbm.at[pl.ds(r, BURST // 128), :], didx)
            for d in gds:
                d.wait()
            sds = []
            for j in range(BURST // 128):
                sds.append(pltpu.async_copy(
                    rows.at[pl.ds(128 * j, 128), :],
                    acc.at[didx.at[j]], ssem, add=True))
            for d in sds:
                d.wait()

        plsc.subcore_barrier()
        w0 = sid * WROWS
        pltpu.sync_copy(acc.at[pl.ds(w0, WROWS), :],
                        r_hbm.at[pl.ds(c * NP + w0, WROWS), :])
        plsc.subcore_barrier()


BPT = B // NW


@functools.partial(
    pl.kernel,
    out_type=jax.ShapeDtypeStruct((B,), jnp.float32),
    compiler_params=_SC_PARAMS,
    mesh=_mesh(),
    scratch_types=[
        pltpu.VMEM((BPT,), jnp.int32),
        pltpu.VMEM((BPT,), jnp.int32),
        pltpu.VMEM((BPT,), jnp.int32),
        pltpu.VMEM((BPT, DC), jnp.float32),
        pltpu.VMEM((BPT, DC), jnp.float32),
        pltpu.VMEM((BPT, DC), jnp.float32),
        pltpu.VMEM((256,), jnp.float32),
        pltpu.VMEM((BPT,), jnp.float32),
        pltpu.SemaphoreType.DMA,
    ],
)
def _sc_score(acc_hbm, uidx_hbm, iidx_hbm, out_hbm, uv, iv, adj, ur, ir, sv,
              tmpf, res, sem):
    cid = lax.axis_index("c")
    sid = lax.axis_index("s")
    wid = sid * NCORE + cid
    base = wid * BPT

    pltpu.sync_copy(uidx_hbm.at[pl.ds(base, BPT)], uv)
    pltpu.sync_copy(iidx_hbm.at[pl.ds(base, BPT)], iv)

    zero = jnp.zeros((16,), jnp.float32)

    @pl.loop(0, BPT)
    def _z(b):
        sv[b, :] = zero

    for c in range(C):
        uoff = jnp.full((16,), c * NP, jnp.int32)
        ioff = jnp.full((16,), c * NP + NU, jnp.int32)
        for j in range(BPT // 16):
            adj[pl.ds(16 * j, 16)] = uv[pl.ds(16 * j, 16)] + uoff
        pltpu.async_copy(acc_hbm.at[adj], ur, sem).wait()
        for j in range(BPT // 16):
            adj[pl.ds(16 * j, 16)] = iv[pl.ds(16 * j, 16)] + ioff
        pltpu.async_copy(acc_hbm.at[adj], ir, sem).wait()

        @pl.loop(0, BPT)
        def _mac(b):
            sv[b, :] = sv[b, :] + ur[b, :] * ir[b, :]

    col = lax.iota(jnp.int32, 16) * 16
    for g in range(BPT // 16):
        for r in range(16):
            plsc.store_scatter(tmpf, [col + r], sv[16 * g + r, :])
        tot = jnp.zeros((16,), jnp.float32)
        for j in range(16):
            tot = tot + tmpf[pl.ds(16 * j, 16)]
        res[pl.ds(16 * g, 16)] = tot * (1.0 / 16.0)

    pltpu.sync_copy(res, out_hbm.at[pl.ds(base, BPT)])


TCBLK = 128


def _tc_init_body(emb_ref, deg_ref, t_ref):
    dinv = lax.rsqrt(jnp.maximum(deg_ref[...], 1.0))
    t_ref[...] = emb_ref[...] * dinv


def _tc_layer_body(r_ref, deg_ref, acc_ref, t_ref, accn_ref):
    dinv = lax.rsqrt(jnp.maximum(deg_ref[...], 1.0))
    h = r_ref[...] * dinv
    accn_ref[...] = acc_ref[...] + h
    t_ref[...] = h * dinv


def _tc_init(embv, degrep):
    grid = FLATR // TCBLK
    bs = pl.BlockSpec((TCBLK, 128), lambda i: (i, 0))
    return pl.pallas_call(
        _tc_init_body,
        grid=(grid,),
        in_specs=[bs, bs],
        out_specs=bs,
        out_shape=jax.ShapeDtypeStruct((FLATR, 128), jnp.float32),
    )(embv, degrep)


def _tc_layer(rv, degrep, accv):
    grid = FLATR // TCBLK
    bs = pl.BlockSpec((TCBLK, 128), lambda i: (i, 0))
    return pl.pallas_call(
        _tc_layer_body,
        grid=(grid,),
        in_specs=[bs, bs, bs],
        out_specs=[bs, bs],
        out_shape=[
            jax.ShapeDtypeStruct((FLATR, 128), jnp.float32),
            jax.ShapeDtypeStruct((FLATR, 128), jnp.float32),
        ],
    )(rv, degrep, accv)


def kernel(user_emb, item_emb, user_idx, item_idx, edge_index):
    u = edge_index[0]
    it = edge_index[1] + NU
    src = jnp.concatenate([u, it])
    dst = jnp.concatenate([it, u])
    src_p = jnp.concatenate([src, jnp.zeros((PAD,), jnp.int32)])
    dst_p = jnp.concatenate(
        [dst, DUMMY + (jnp.arange(PAD, dtype=jnp.int32) % (NP - N))])
    src2d = src_p.reshape(E2P // 128, 128)
    dst2d = dst_p.reshape(E2P // 128, 128)

    embc = (jnp.concatenate([user_emb, item_emb], axis=0)
            .reshape(N, C, DC).transpose(1, 0, 2))
    embc = jnp.pad(embc, ((0, 0), (0, NP - N), (0, 0))).reshape(C * NP, DC)
    embv = embc.reshape(FLATR, 128)

    deg2 = _sc_degree(dst2d)
    deg = (deg2[0] + deg2[1]).reshape(-1)[:NP]
    degrep = jnp.broadcast_to(deg[None, :, None], (C, NP, DC)).reshape(FLATR, 128)

    t = _tc_init(embv, degrep)
    acc = embv
    for _ in range(3):
        r = _sc_layer(t.reshape(C * NP, DC), src2d, dst2d)
        t, acc = _tc_layer(r.reshape(FLATR, 128), degrep, acc)

    return _sc_score(acc.reshape(C * NP, DC), user_idx, item_idx)


# speedup vs baseline: 2.0118x; 1.0751x over previous
import functools

import jax
import jax.numpy as jnp
from jax import lax
from jax.experimental import pallas as pl
from jax.experimental.pallas import tpu as pltpu
from jax.experimental.pallas import tpu_sc as plsc

NU = 50000
NI = 50000
N = NU + NI
D = 64
C = 4
DC = D // C
E2 = 2500000
B = 4096

NCORE = 2
NSUB = 16
NW = NCORE * NSUB

BURST = 1536
E2P = 2506752
PAD = E2P - E2
EPT = E2P // NSUB
NBURST = EPT // BURST
DUMMY = N
NP = 100096
WROWS = NP // NSUB

DEG_SLICE = 6400
DEG_ROWS = NSUB * DEG_SLICE
DEG_BURST = 512
DEG_EPT = E2P // NW
DEG_NBURST = DEG_EPT // DEG_BURST

FLATR = C * NP * DC // 128


def _mesh():
    return plsc.VectorSubcoreMesh(core_axis_name="c", subcore_axis_name="s",
                                  num_cores=NCORE, num_subcores=NSUB)


_SC_PARAMS = pltpu.CompilerParams(use_tc_tiling_on_sc=False,
                                  needs_layout_passes=False)


@functools.partial(
    pl.kernel,
    out_type=jax.ShapeDtypeStruct((NCORE, NSUB, DEG_SLICE), jnp.float32),
    compiler_params=_SC_PARAMS,
    mesh=_mesh(),
    scratch_types=[
        pltpu.VMEM((DEG_BURST // 128, 128), jnp.int32),
        pltpu.VMEM((128,), jnp.float32),
        pltpu.VMEM((DEG_SLICE,), jnp.float32),
        pltpu.VMEM_SHARED((DEG_ROWS,), jnp.float32),
        pltpu.SemaphoreType.DMA,
    ],
)
def _sc_degree(dst_hbm, deg_hbm, didx, ones, zeros, hist, sem):
    cid = lax.axis_index("c")
    sid = lax.axis_index("s")

    one = jnp.full((16,), 1.0, jnp.float32)
    zero = jnp.zeros((16,), jnp.float32)
    for i in range(128 // 16):
        ones[pl.ds(16 * i, 16)] = one

    @pl.loop(0, DEG_SLICE // 16)
    def _z(i):
        zeros[pl.ds(16 * i, 16)] = zero

    pltpu.sync_copy(zeros, hist.at[pl.ds(sid * DEG_SLICE, DEG_SLICE)])
    plsc.subcore_barrier()

    row0 = (cid * NSUB + sid) * (DEG_EPT // 128)

    @pl.loop(0, DEG_NBURST)
    def _burst(b):
        r = row0 + b * (DEG_BURST // 128)
        pltpu.sync_copy(dst_hbm.at[pl.ds(r, DEG_BURST // 128), :], didx)
        descs = []
        for j in range(DEG_BURST // 128):
            descs.append(
                pltpu.async_copy(ones, hist.at[didx.at[j]], sem, add=True))
        for d in descs:
            d.wait()

    plsc.subcore_barrier()
    pltpu.sync_copy(hist.at[pl.ds(sid * DEG_SLICE, DEG_SLICE)],
                    deg_hbm.at[cid, sid])


@functools.partial(
    pl.kernel,
    out_type=jax.ShapeDtypeStruct((C * NP, DC), jnp.float32),
    compiler_params=_SC_PARAMS,
    mesh=_mesh(),
    scratch_types=[
        pltpu.VMEM((BURST // 128, 128), jnp.int32),
        pltpu.VMEM((BURST // 128, 128), jnp.int32),
        pltpu.VMEM((BURST, DC), jnp.float32),
        pltpu.VMEM((WROWS // 64, DC), jnp.float32),
        pltpu.VMEM_SHARED((NP, DC), jnp.float32),
        pltpu.SemaphoreType.DMA,
        pltpu.SemaphoreType.DMA,
    ],
)
def _sc_layer(t_hbm, s_hbm, d_hbm, r_hbm, sidx, didx, rows, zeros, acc,
              gsem, ssem):
    cid = lax.axis_index("c")
    sid = lax.axis_index("s")

    zero = jnp.zeros((16,), jnp.float32)

    @pl.loop(0, WROWS // 64)
    def _z(i):
        zeros[i, :] = zero

    row0 = sid * (EPT // 128)

    for cc in range(2):
        c = 2 * cid + cc
        off = jnp.full((16,), c * NP, jnp.int32)

        z0 = sid * WROWS
        for zb in range(64):
            pltpu.sync_copy(
                zeros,
                acc.at[pl.ds(z0 + zb * (WROWS // 64), WROWS // 64), :])
        plsc.subcore_barrier()

        @pl.loop(0, NBURST)
        def _burst(b):
            r = row0 + b * (BURST // 128)
            pltpu.sync_copy(s_hbm.at[pl.ds(r, BURST // 128), :], sidx)
            for i in range(BURST // 128):
                for j in range(128 // 16):
                    sl = pl.ds(16 * j, 16)
                    sidx[i, sl] = sidx[i, sl] + off
            gds = []
            for j in range(BURST // 128):
                gds.append(pltpu.async_copy(
                    t_hbm.at[sidx.at[j]],
                    rows.at[pl.ds(128 * j, 128), :], gsem))
            pltpu.sync_copy(d_hbm.at[pl.ds(r, BURST // 128), :], didx)
            for d in gds:
                d.wait()
            sds = []
            for j in range(BURST // 128):
                sds.append(pltpu.async_copy(
                    rows.at[pl.ds(128 * j, 128), :],
                    acc.at[didx.at[j]], ssem, add=True))
            for d in sds:
                d.wait()

        plsc.subcore_barrier()
        w0 = sid * WROWS
        pltpu.sync_copy(acc.at[pl.ds(w0, WROWS), :],
                        r_hbm.at[pl.ds(c * NP + w0, WROWS), :])
        plsc.subcore_barrier()


BPT = B // NW


@functools.partial(
    pl.kernel,
    out_type=jax.ShapeDtypeStruct((B,), jnp.float32),
    compiler_params=_SC_PARAMS,
    mesh=_mesh(),
    scratch_types=[
        pltpu.VMEM((BPT,), jnp.int32),
        pltpu.VMEM((BPT,), jnp.int32),
        pltpu.VMEM((BPT,), jnp.int32),
        pltpu.VMEM((BPT, DC), jnp.float32),
        pltpu.VMEM((BPT, DC), jnp.float32),
        pltpu.VMEM((BPT, DC), jnp.float32),
        pltpu.VMEM((256,), jnp.float32),
        pltpu.VMEM((BPT,), jnp.float32),
        pltpu.SemaphoreType.DMA,
    ],
)
def _sc_score(acc_hbm, uidx_hbm, iidx_hbm, out_hbm, uv, iv, adj, ur, ir, sv,
              tmpf, res, sem):
    cid = lax.axis_index("c")
    sid = lax.axis_index("s")
    wid = sid * NCORE + cid
    base = wid * BPT

    pltpu.sync_copy(uidx_hbm.at[pl.ds(base, BPT)], uv)
    pltpu.sync_copy(iidx_hbm.at[pl.ds(base, BPT)], iv)

    zero = jnp.zeros((16,), jnp.float32)

    @pl.loop(0, BPT)
    def _z(b):
        sv[b, :] = zero

    for c in range(C):
        uoff = jnp.full((16,), c * NP, jnp.int32)
        ioff = jnp.full((16,), c * NP + NU, jnp.int32)
        for j in range(BPT // 16):
            adj[pl.ds(16 * j, 16)] = uv[pl.ds(16 * j, 16)] + uoff
        pltpu.async_copy(acc_hbm.at[adj], ur, sem).wait()
        for j in range(BPT // 16):
            adj[pl.ds(16 * j, 16)] = iv[pl.ds(16 * j, 16)] + ioff
        pltpu.async_copy(acc_hbm.at[adj], ir, sem).wait()

        @pl.loop(0, BPT)
        def _mac(b):
            sv[b, :] = sv[b, :] + ur[b, :] * ir[b, :]

    col = lax.iota(jnp.int32, 16) * 16
    for g in range(BPT // 16):
        for r in range(16):
            plsc.store_scatter(tmpf, [col + r], sv[16 * g + r, :])
        tot = jnp.zeros((16,), jnp.float32)
        for j in range(16):
            tot = tot + tmpf[pl.ds(16 * j, 16)]
        res[pl.ds(16 * g, 16)] = tot * (1.0 / 16.0)

    pltpu.sync_copy(res, out_hbm.at[pl.ds(base, BPT)])


TCBLK = 128


def _tc_init_body(emb_ref, deg_ref, t_ref):
    dinv = lax.rsqrt(jnp.maximum(deg_ref[...], 1.0))
    t_ref[...] = emb_ref[...] * dinv


def _tc_layer_body(r_ref, deg_ref, acc_ref, t_ref, accn_ref):
    dinv = lax.rsqrt(jnp.maximum(deg_ref[...], 1.0))
    h = r_ref[...] * dinv
    accn_ref[...] = acc_ref[...] + h
    t_ref[...] = h * dinv


def _tc_init(embv, degrep):
    grid = FLATR // TCBLK
    bs = pl.BlockSpec((TCBLK, 128), lambda i: (i, 0))
    return pl.pallas_call(
        _tc_init_body,
        grid=(grid,),
        in_specs=[bs, bs],
        out_specs=bs,
        out_shape=jax.ShapeDtypeStruct((FLATR, 128), jnp.float32),
    )(embv, degrep)


def _tc_layer(rv, degrep, accv):
    grid = FLATR // TCBLK
    bs = pl.BlockSpec((TCBLK, 128), lambda i: (i, 0))
    return pl.pallas_call(
        _tc_layer_body,
        grid=(grid,),
        in_specs=[bs, bs, bs],
        out_specs=[bs, bs],
        out_shape=[
            jax.ShapeDtypeStruct((FLATR, 128), jnp.float32),
            jax.ShapeDtypeStruct((FLATR, 128), jnp.float32),
        ],
    )(rv, degrep, accv)


def kernel(user_emb, item_emb, user_idx, item_idx, edge_index):
    u = edge_index[0]
    it = edge_index[1] + NU
    src = jnp.concatenate([u, it])
    dst = jnp.concatenate([it, u])
    src_p = jnp.concatenate([src, jnp.zeros((PAD,), jnp.int32)])
    dst_p = jnp.concatenate(
        [dst, DUMMY + (jnp.arange(PAD, dtype=jnp.int32) % (NP - N))])
    src2d = src_p.reshape(E2P // 128, 128)
    dst2d = dst_p.reshape(E2P // 128, 128)

    embc = (jnp.concatenate([user_emb, item_emb], axis=0)
            .reshape(N, C, DC).transpose(1, 0, 2))
    embc = jnp.pad(embc, ((0, 0), (0, NP - N), (0, 0))).reshape(C * NP, DC)
    embv = embc.reshape(FLATR, 128)

    deg2 = _sc_degree(dst2d)
    deg = (deg2[0] + deg2[1]).reshape(-1)[:NP]
    degrep = jnp.broadcast_to(deg[None, :, None], (C, NP, DC)).reshape(FLATR, 128)

    t = _tc_init(embv, degrep)
    acc = embv
    for _ in range(3):
        r = _sc_layer(t.reshape(C * NP, DC), src2d, dst2d)
        t, acc = _tc_layer(r.reshape(FLATR, 128), degrep, acc)

    return _sc_score(acc.reshape(C * NP, DC), user_idx, item_idx)
